# baseline (speedup 1.0000x reference)
import functools

import jax
import jax.numpy as jnp
import numpy as np
from jax import lax
from jax.experimental import pallas as pl
from jax.experimental.pallas import tpu as pltpu
from jax.experimental.pallas import tpu_sc as plsc

_N = 20000
_NC, _NS = 2, 16
_NW = _NC * _NS
_RW = 624
_NG = _RW // 16
_TAIL_BASE = _NW * _RW

_HALF_PI = np.float32(np.pi / 2.0)

_mesh = plsc.VectorSubcoreMesh(core_axis_name="c", subcore_axis_name="s")


def _do_group(r, pb_r, ps_r, po_r, g_r, tb_r, so_r, orient_v, d0, d1, d2):
    lane = lax.iota(jnp.int32, 16)
    po_v = po_r[pl.ds(r, 16)]
    mk = jnp.where(po_v >= 0.9, 1.0, 0.0).astype(jnp.float32)
    b7 = lane * 7 + r * 7
    b4 = lane * 4 + r * 4
    b2 = lane * 2 + r * 2
    x0 = plsc.load_gather(pb_r, [b7])
    s0 = 1.0 / (1.0 + jnp.exp(-x0))
    plsc.store_scatter(tb_r, [b7], (orient_v + s0 * _HALF_PI) * mk)
    for c in (1, 2):
        x = plsc.load_gather(pb_r, [b7 + c])
        gv = plsc.load_gather(g_r, [b2 + (c - 1)])
        s = 1.0 / (1.0 + jnp.exp(-x))
        plsc.store_scatter(tb_r, [b7 + c], (s + gv + 0.5) * mk)
    x3 = plsc.load_gather(pb_r, [b7 + 3])
    s3 = 1.0 / (1.0 + jnp.exp(-x3))
    plsc.store_scatter(tb_r, [b7 + 3], s3 * mk)
    for c, dv in ((4, d0), (5, d1), (6, d2)):
        x = plsc.load_gather(pb_r, [b7 + c])
        plsc.store_scatter(tb_r, [b7 + c], dv * jnp.exp(x) * mk)
    for c in range(4):
        sv = plsc.load_gather(ps_r, [b4 + c])
        plsc.store_scatter(so_r, [b4 + c], sv * mk)


@functools.partial(
    pl.kernel,
    out_type=[
        jax.ShapeDtypeStruct((_N * 7,), jnp.float32),
        jax.ShapeDtypeStruct((_N * 4,), jnp.float32),
        jax.ShapeDtypeStruct((_N * 7,), jnp.float32),
        jax.ShapeDtypeStruct((_N * 4,), jnp.float32),
    ],
    mesh=_mesh,
    compiler_params=pltpu.CompilerParams(needs_layout_passes=False),
    scratch_types=[
        pltpu.VMEM((80,), jnp.float32),
        pltpu.VMEM((_RW * 7,), jnp.float32),
        pltpu.VMEM((_RW * 4,), jnp.float32),
        pltpu.VMEM((_RW,), jnp.float32),
        pltpu.VMEM((_RW * 2,), jnp.float32),
        pltpu.VMEM((_RW * 7,), jnp.float32),
        pltpu.VMEM((_RW * 4,), jnp.float32),
        pltpu.VMEM((112,), jnp.float32),
        pltpu.VMEM((64,), jnp.float32),
        pltpu.VMEM((16,), jnp.float32),
        pltpu.VMEM((32,), jnp.float32),
        pltpu.VMEM((112,), jnp.float32),
        pltpu.VMEM((64,), jnp.float32),
    ],
)
def _sc_fwd(pb1, ps1, po1, g1, pb2, ps2, po2, g2, cst,
            ob1, os1, ob2, os2,
            c_vm, pb_vm, ps_vm, po_vm, g_vm, tb_vm, so_vm,
            tpb, tps, tpo, tg, ttb, tso):
    wid = lax.axis_index("s") * _NC + lax.axis_index("c")
    pltpu.sync_copy(cst, c_vm)
    d0 = c_vm[pl.ds(32, 16)]
    d1 = c_vm[pl.ds(48, 16)]
    d2 = c_vm[pl.ds(64, 16)]
    base = wid * _RW
    heads = ((pb1, ps1, po1, g1, ob1, os1), (pb2, ps2, po2, g2, ob2, os2))
    for h, (pbh, psh, poh, gh, obh, osh) in enumerate(heads):
        orient_v = c_vm[pl.ds(h * 16, 16)]
        pltpu.sync_copy(pbh.at[pl.ds(base * 7, _RW * 7)], pb_vm)
        pltpu.sync_copy(psh.at[pl.ds(base * 4, _RW * 4)], ps_vm)
        pltpu.sync_copy(poh.at[pl.ds(base, _RW)], po_vm)
        pltpu.sync_copy(gh.at[pl.ds(base * 2, _RW * 2)], g_vm)

        def body(i, carry):
            _do_group(i * 16, pb_vm, ps_vm, po_vm, g_vm, tb_vm, so_vm,
                      orient_v, d0, d1, d2)
            return carry

        lax.fori_loop(0, _NG, body, 0)

        pltpu.sync_copy(tb_vm, obh.at[pl.ds(base * 7, _RW * 7)])
        pltpu.sync_copy(so_vm, osh.at[pl.ds(base * 4, _RW * 4)])

        trow = _TAIL_BASE + wid * 16

        @pl.when(wid < 2)
        def _tail():
            pltpu.sync_copy(pbh.at[pl.ds(trow * 7, 112)], tpb)
            pltpu.sync_copy(psh.at[pl.ds(trow * 4, 64)], tps)
            pltpu.sync_copy(poh.at[pl.ds(trow, 16)], tpo)
            pltpu.sync_copy(gh.at[pl.ds(trow * 2, 32)], tg)
            _do_group(0, tpb, tps, tpo, tg, ttb, tso, orient_v, d0, d1, d2)
            pltpu.sync_copy(ttb, obh.at[pl.ds(trow * 7, 112)])
            pltpu.sync_copy(tso, osh.at[pl.ds(trow * 4, 64)])


def kernel(pred_bboxes1, pred_class_scores1, pred_objectness1, pred_bboxes_grid1,
           pred_bboxes2, pred_class_scores2, pred_objectness2, pred_bboxes_grid2,
           anchor_orients, anchor_dims):
    cst = jnp.broadcast_to(
        jnp.concatenate([anchor_orients, anchor_dims])[:, None], (5, 16)
    ).reshape(-1)
    ob1, os1, ob2, os2 = _sc_fwd(
        pred_bboxes1.reshape(-1), pred_class_scores1.reshape(-1),
        pred_objectness1, pred_bboxes_grid1.reshape(-1),
        pred_bboxes2.reshape(-1), pred_class_scores2.reshape(-1),
        pred_objectness2, pred_bboxes_grid2.reshape(-1),
        cst,
    )
    return (ob1.reshape(_N, 7), os1.reshape(_N, 4),
            ob2.reshape(_N, 7), os2.reshape(_N, 4))


# speedup vs baseline: 7.5513x; 7.5513x over previous
import functools

import jax
import jax.numpy as jnp
import numpy as np
from jax import lax
from jax.experimental import pallas as pl
from jax.experimental.pallas import tpu as pltpu
from jax.experimental.pallas import tpu_sc as plsc

_N = 20000
_NC, _NS = 2, 16
_NW = _NC * _NS
_LW = 640
_LT = _N - 31 * _LW

_HALF_PI = np.float32(np.pi / 2.0)


def _broadcast_lane(buf, i):
    return plsc.load_gather(buf, [jnp.full((16,), i, dtype=jnp.int32)])


def _do_group(s, pb_b, ps_b, po_b, g_b, tb_b, so_b, orient_v, d0, d1, d2):
    po_v = po_b[pl.ds(s, 16)]
    mk = jnp.where(po_v >= 0.9, 1.0, 0.0).astype(jnp.float32)
    x0 = pb_b[0, pl.ds(s, 16)]
    s0 = 1.0 / (1.0 + jnp.exp(-x0))
    tb_b[0, pl.ds(s, 16)] = (orient_v + s0 * _HALF_PI) * mk
    for c in (1, 2):
        x = pb_b[c, pl.ds(s, 16)]
        gv = g_b[c - 1, pl.ds(s, 16)]
        sg = 1.0 / (1.0 + jnp.exp(-x))
        tb_b[c, pl.ds(s, 16)] = (sg + gv + 0.5) * mk
    x3 = pb_b[3, pl.ds(s, 16)]
    s3 = 1.0 / (1.0 + jnp.exp(-x3))
    tb_b[3, pl.ds(s, 16)] = s3 * mk
    for c, dv in ((4, d0), (5, d1), (6, d2)):
        x = pb_b[c, pl.ds(s, 16)]
        tb_b[c, pl.ds(s, 16)] = dv * jnp.exp(x) * mk
    for c in range(4):
        so_b[c, pl.ds(s, 16)] = ps_b[c, pl.ds(s, 16)] * mk


_mesh = plsc.VectorSubcoreMesh(core_axis_name="c", subcore_axis_name="s")


@functools.partial(
    pl.kernel,
    out_type=[
        jax.ShapeDtypeStruct((7, _N), jnp.float32),
        jax.ShapeDtypeStruct((4, _N), jnp.float32),
        jax.ShapeDtypeStruct((7, _N), jnp.float32),
        jax.ShapeDtypeStruct((4, _N), jnp.float32),
    ],
    mesh=_mesh,
    compiler_params=pltpu.CompilerParams(needs_layout_passes=False),
    scratch_types=[
        pltpu.VMEM((16,), jnp.float32),
        pltpu.VMEM((16,), jnp.float32),
        pltpu.VMEM((7, _LW), jnp.float32),
        pltpu.VMEM((4, _LW), jnp.float32),
        pltpu.VMEM((_LW,), jnp.float32),
        pltpu.VMEM((2, _LW), jnp.float32),
        pltpu.VMEM((7, _LW), jnp.float32),
        pltpu.VMEM((4, _LW), jnp.float32),
        pltpu.VMEM((7, _LW), jnp.float32),
        pltpu.VMEM((4, _LW), jnp.float32),
        pltpu.VMEM((_LW,), jnp.float32),
        pltpu.VMEM((2, _LW), jnp.float32),
        pltpu.VMEM((7, _LW), jnp.float32),
        pltpu.VMEM((4, _LW), jnp.float32),
        pltpu.VMEM((7, 32), jnp.float32),
        pltpu.VMEM((4, 32), jnp.float32),
        pltpu.VMEM((32,), jnp.float32),
        pltpu.VMEM((2, 32), jnp.float32),
        pltpu.VMEM((7, 32), jnp.float32),
        pltpu.VMEM((4, 32), jnp.float32),
        pltpu.SemaphoreType.DMA,
        pltpu.SemaphoreType.DMA,
        pltpu.SemaphoreType.DMA,
    ],
)
def _sc_fwd(pb1, ps1, po1, g1, pb2, ps2, po2, g2, orients, dims,
            ob1, os1, ob2, os2,
            c_vm, d_vm,
            pb_v1, ps_v1, po_v1, g_v1, tb_v1, so_v1,
            pb_v2, ps_v2, po_v2, g_v2, tb_v2, so_v2,
            tpb, tps, tpo, tg, ttb, tso,
            sem1, sem2, semo):
    wid = lax.axis_index("s") * _NC + lax.axis_index("c")
    pltpu.sync_copy(orients, c_vm.at[pl.ds(8, 2)])
    pltpu.sync_copy(dims, d_vm.at[pl.ds(8, 3)])
    d0 = _broadcast_lane(d_vm, 8)
    d1 = _broadcast_lane(d_vm, 9)
    d2 = _broadcast_lane(d_vm, 10)
    orients_v = (_broadcast_lane(c_vm, 8), _broadcast_lane(c_vm, 9))

    heads = (
        (pb1, ps1, po1, g1, ob1, os1, pb_v1, ps_v1, po_v1, g_v1, tb_v1, so_v1, sem1),
        (pb2, ps2, po2, g2, ob2, os2, pb_v2, ps_v2, po_v2, g_v2, tb_v2, so_v2, sem2),
    )

    def run(l0, nl, ng):
        in_cps = []
        for (pbh, psh, poh, gh, _, _, pb_v, ps_v, po_v, g_v, _, _, sem) in heads:
            in_cps.append([
                pltpu.async_copy(pbh.at[:, pl.ds(l0, nl)], pb_v.at[:, pl.ds(0, nl)], sem),
                pltpu.async_copy(psh.at[:, pl.ds(l0, nl)], ps_v.at[:, pl.ds(0, nl)], sem),
                pltpu.async_copy(poh.at[pl.ds(l0, nl)], po_v.at[pl.ds(0, nl)], sem),
                pltpu.async_copy(gh.at[:, pl.ds(l0, nl)], g_v.at[:, pl.ds(0, nl)], sem),
            ])
        out_cps = []
        for h, (_, _, _, _, obh, osh, pb_v, ps_v, po_v, g_v, tb_v, so_v, _) in enumerate(heads):
            for cp in in_cps[h]:
                cp.wait()

            def body(i, carry):
                _do_group(i * 16, pb_v, ps_v, po_v, g_v, tb_v, so_v,
                          orients_v[h], d0, d1, d2)
                return carry

            lax.fori_loop(0, ng, body, 0)
            out_cps.append(
                pltpu.async_copy(tb_v.at[:, pl.ds(0, nl)], obh.at[:, pl.ds(l0, nl)], semo))
            out_cps.append(
                pltpu.async_copy(so_v.at[:, pl.ds(0, nl)], osh.at[:, pl.ds(l0, nl)], semo))
        for cp in out_cps:
            cp.wait()

    @pl.when(wid < 31)
    def _main():
        run(wid * _LW, _LW, _LW // 16)

    @pl.when(wid == 31)
    def _tail():
        run(31 * _LW, 128, 8)
        t0 = _N - 32
        for h, (pbh, psh, poh, gh, obh, osh, *_) in enumerate(heads):
            cps = [
                pltpu.async_copy(pbh.at[:, pl.ds(t0, 32)], tpb, sem1),
                pltpu.async_copy(psh.at[:, pl.ds(t0, 32)], tps, sem1),
                pltpu.async_copy(poh.at[pl.ds(t0, 32)], tpo, sem1),
                pltpu.async_copy(gh.at[:, pl.ds(t0, 32)], tg, sem1),
            ]
            for cp in cps:
                cp.wait()
            for s in (0, 16):
                _do_group(s, tpb, tps, tpo, tg, ttb, tso,
                          orients_v[h], d0, d1, d2)
            ocps = [
                pltpu.async_copy(ttb, obh.at[:, pl.ds(t0, 32)], semo),
                pltpu.async_copy(tso, osh.at[:, pl.ds(t0, 32)], semo),
            ]
            for cp in ocps:
                cp.wait()


def kernel(pred_bboxes1, pred_class_scores1, pred_objectness1, pred_bboxes_grid1,
           pred_bboxes2, pred_class_scores2, pred_objectness2, pred_bboxes_grid2,
           anchor_orients, anchor_dims):
    ob1, os1, ob2, os2 = _sc_fwd(
        pred_bboxes1.T, pred_class_scores1.T, pred_objectness1, pred_bboxes_grid1.T,
        pred_bboxes2.T, pred_class_scores2.T, pred_objectness2, pred_bboxes_grid2.T,
        anchor_orients, anchor_dims,
    )
    return (ob1.T, os1.T, ob2.T, os2.T)


# speedup vs baseline: 7.5835x; 1.0043x over previous
import functools

import jax
import jax.numpy as jnp
import numpy as np
from jax import lax
from jax.experimental import pallas as pl
from jax.experimental.pallas import tpu as pltpu
from jax.experimental.pallas import tpu_sc as plsc

_N = 20000
_NC, _NS = 2, 16
_NW = _NC * _NS
_LW = 640
_LT = _N - 31 * _LW

_HALF_PI = np.float32(np.pi / 2.0)


def _broadcast_lane(buf, i):
    return plsc.load_gather(buf, [jnp.full((16,), i, dtype=jnp.int32)])


def _do_group(s, pb_b, ps_b, po_b, g_b, tb_b, so_b, orient_v, d0, d1, d2):
    po_v = po_b[pl.ds(s, 16)]
    mk = jnp.where(po_v >= 0.9, 1.0, 0.0).astype(jnp.float32)
    x0 = pb_b[0, pl.ds(s, 16)]
    s0 = 1.0 / (1.0 + jnp.exp(-x0))
    tb_b[0, pl.ds(s, 16)] = (orient_v + s0 * _HALF_PI) * mk
    for c in (1, 2):
        x = pb_b[c, pl.ds(s, 16)]
        gv = g_b[c - 1, pl.ds(s, 16)]
        sg = 1.0 / (1.0 + jnp.exp(-x))
        tb_b[c, pl.ds(s, 16)] = (sg + gv + 0.5) * mk
    x3 = pb_b[3, pl.ds(s, 16)]
    s3 = 1.0 / (1.0 + jnp.exp(-x3))
    tb_b[3, pl.ds(s, 16)] = s3 * mk
    for c, dv in ((4, d0), (5, d1), (6, d2)):
        x = pb_b[c, pl.ds(s, 16)]
        tb_b[c, pl.ds(s, 16)] = dv * jnp.exp(x) * mk
    for c in range(4):
        so_b[c, pl.ds(s, 16)] = ps_b[c, pl.ds(s, 16)] * mk


_mesh = plsc.VectorSubcoreMesh(core_axis_name="c", subcore_axis_name="s")


@functools.partial(
    pl.kernel,
    out_type=[
        jax.ShapeDtypeStruct((7, _N), jnp.float32),
        jax.ShapeDtypeStruct((4, _N), jnp.float32),
        jax.ShapeDtypeStruct((7, _N), jnp.float32),
        jax.ShapeDtypeStruct((4, _N), jnp.float32),
    ],
    mesh=_mesh,
    compiler_params=pltpu.CompilerParams(needs_layout_passes=False),
    scratch_types=[
        pltpu.VMEM((16,), jnp.float32),
        pltpu.VMEM((16,), jnp.float32),
        pltpu.VMEM((7, _LW), jnp.float32),
        pltpu.VMEM((4, _LW), jnp.float32),
        pltpu.VMEM((_LW,), jnp.float32),
        pltpu.VMEM((2, _LW), jnp.float32),
        pltpu.VMEM((7, _LW), jnp.float32),
        pltpu.VMEM((4, _LW), jnp.float32),
        pltpu.VMEM((7, _LW), jnp.float32),
        pltpu.VMEM((4, _LW), jnp.float32),
        pltpu.VMEM((_LW,), jnp.float32),
        pltpu.VMEM((2, _LW), jnp.float32),
        pltpu.VMEM((7, _LW), jnp.float32),
        pltpu.VMEM((4, _LW), jnp.float32),
        pltpu.VMEM((7, 32), jnp.float32),
        pltpu.VMEM((4, 32), jnp.float32),
        pltpu.VMEM((32,), jnp.float32),
        pltpu.VMEM((2, 32), jnp.float32),
        pltpu.VMEM((7, 32), jnp.float32),
        pltpu.VMEM((4, 32), jnp.float32),
        pltpu.SemaphoreType.DMA,
        pltpu.SemaphoreType.DMA,
        pltpu.SemaphoreType.DMA,
    ],
)
def _sc_fwd(pb1, ps1, po1, g1, pb2, ps2, po2, g2, orients, dims,
            ob1, os1, ob2, os2,
            c_vm, d_vm,
            pb_v1, ps_v1, po_v1, g_v1, tb_v1, so_v1,
            pb_v2, ps_v2, po_v2, g_v2, tb_v2, so_v2,
            tpb, tps, tpo, tg, ttb, tso,
            sem1, sem2, semo):
    wid = lax.axis_index("s") * _NC + lax.axis_index("c")
    pltpu.sync_copy(orients, c_vm.at[pl.ds(8, 2)])
    pltpu.sync_copy(dims, d_vm.at[pl.ds(8, 3)])
    d0 = _broadcast_lane(d_vm, 8)
    d1 = _broadcast_lane(d_vm, 9)
    d2 = _broadcast_lane(d_vm, 10)
    orients_v = (_broadcast_lane(c_vm, 8), _broadcast_lane(c_vm, 9))

    heads = (
        (pb1, ps1, po1, g1, ob1, os1, pb_v1, ps_v1, po_v1, g_v1, tb_v1, so_v1, sem1),
        (pb2, ps2, po2, g2, ob2, os2, pb_v2, ps_v2, po_v2, g_v2, tb_v2, so_v2, sem2),
    )

    def run(l0, nl, ng):
        in_cps = []
        for (pbh, psh, poh, gh, _, _, pb_v, ps_v, po_v, g_v, _, _, sem) in heads:
            in_cps.append([
                pltpu.async_copy(pbh.at[:, pl.ds(l0, nl)], pb_v.at[:, pl.ds(0, nl)], sem),
                pltpu.async_copy(psh.at[:, pl.ds(l0, nl)], ps_v.at[:, pl.ds(0, nl)], sem),
                pltpu.async_copy(poh.at[pl.ds(l0, nl)], po_v.at[pl.ds(0, nl)], sem),
                pltpu.async_copy(gh.at[:, pl.ds(l0, nl)], g_v.at[:, pl.ds(0, nl)], sem),
            ])
        out_cps = []
        for h, (_, _, _, _, obh, osh, pb_v, ps_v, po_v, g_v, tb_v, so_v, _) in enumerate(heads):
            for cp in in_cps[h]:
                cp.wait()

            @plsc.parallel_loop(0, ng * 16, step=16, unroll=4)
            def body(s):
                _do_group(s, pb_v, ps_v, po_v, g_v, tb_v, so_v,
                          orients_v[h], d0, d1, d2)
            out_cps.append(
                pltpu.async_copy(tb_v.at[:, pl.ds(0, nl)], obh.at[:, pl.ds(l0, nl)], semo))
            out_cps.append(
                pltpu.async_copy(so_v.at[:, pl.ds(0, nl)], osh.at[:, pl.ds(l0, nl)], semo))
        for cp in out_cps:
            cp.wait()

    @pl.when(wid < 31)
    def _main():
        run(wid * _LW, _LW, _LW // 16)

    @pl.when(wid == 31)
    def _tail():
        run(31 * _LW, 128, 8)
        t0 = _N - 32
        for h, (pbh, psh, poh, gh, obh, osh, *_) in enumerate(heads):
            cps = [
                pltpu.async_copy(pbh.at[:, pl.ds(t0, 32)], tpb, sem1),
                pltpu.async_copy(psh.at[:, pl.ds(t0, 32)], tps, sem1),
                pltpu.async_copy(poh.at[pl.ds(t0, 32)], tpo, sem1),
                pltpu.async_copy(gh.at[:, pl.ds(t0, 32)], tg, sem1),
            ]
            for cp in cps:
                cp.wait()
            for s in (0, 16):
                _do_group(s, tpb, tps, tpo, tg, ttb, tso,
                          orients_v[h], d0, d1, d2)
            ocps = [
                pltpu.async_copy(ttb, obh.at[:, pl.ds(t0, 32)], semo),
                pltpu.async_copy(tso, osh.at[:, pl.ds(t0, 32)], semo),
            ]
            for cp in ocps:
                cp.wait()


def kernel(pred_bboxes1, pred_class_scores1, pred_objectness1, pred_bboxes_grid1,
           pred_bboxes2, pred_class_scores2, pred_objectness2, pred_bboxes_grid2,
           anchor_orients, anchor_dims):
    ob1, os1, ob2, os2 = _sc_fwd(
        pred_bboxes1.T, pred_class_scores1.T, pred_objectness1, pred_bboxes_grid1.T,
        pred_bboxes2.T, pred_class_scores2.T, pred_objectness2, pred_bboxes_grid2.T,
        anchor_orients, anchor_dims,
    )
    return (ob1.T, os1.T, ob2.T, os2.T)


# speedup vs baseline: 8.4780x; 1.1180x over previous
import functools

import jax
import jax.numpy as jnp
import numpy as np
from jax import lax
from jax.experimental import pallas as pl
from jax.experimental.pallas import tpu as pltpu
from jax.experimental.pallas import tpu_sc as plsc

_N = 20000
_NC, _NS = 2, 16
_NW = _NC * _NS
_LW = 640
_LT = _N - 31 * _LW

_HALF_PI = np.float32(np.pi / 2.0)


def _broadcast_lane(buf, i):
    return plsc.load_gather(buf, [jnp.full((16,), i, dtype=jnp.int32)])


def _do_group(s, pb_b, ps_b, po_b, g_b, tb_b, so_b, orient_v, d0, d1, d2):
    po_v = po_b[pl.ds(s, 16)]
    mk = jnp.where(po_v >= 0.9, 1.0, 0.0).astype(jnp.float32)
    x0 = pb_b[0, pl.ds(s, 16)]
    s0 = 1.0 / (1.0 + jnp.exp(-x0))
    tb_b[0, pl.ds(s, 16)] = (orient_v + s0 * _HALF_PI) * mk
    for c in (1, 2):
        x = pb_b[c, pl.ds(s, 16)]
        gv = g_b[c - 1, pl.ds(s, 16)]
        sg = 1.0 / (1.0 + jnp.exp(-x))
        tb_b[c, pl.ds(s, 16)] = (sg + gv + 0.5) * mk
    x3 = pb_b[3, pl.ds(s, 16)]
    s3 = 1.0 / (1.0 + jnp.exp(-x3))
    tb_b[3, pl.ds(s, 16)] = s3 * mk
    for c, dv in ((4, d0), (5, d1), (6, d2)):
        x = pb_b[c, pl.ds(s, 16)]
        tb_b[c, pl.ds(s, 16)] = dv * jnp.exp(x) * mk
    for c in range(4):
        so_b[c, pl.ds(s, 16)] = ps_b[c, pl.ds(s, 16)] * mk


_mesh = plsc.VectorSubcoreMesh(core_axis_name="c", subcore_axis_name="s")


@functools.partial(
    pl.kernel,
    out_type=[
        jax.ShapeDtypeStruct((7, _N), jnp.float32),
        jax.ShapeDtypeStruct((4, _N), jnp.float32),
        jax.ShapeDtypeStruct((7, _N), jnp.float32),
        jax.ShapeDtypeStruct((4, _N), jnp.float32),
    ],
    mesh=_mesh,
    compiler_params=pltpu.CompilerParams(needs_layout_passes=False),
    scratch_types=[
        pltpu.VMEM((16,), jnp.float32),
        pltpu.VMEM((16,), jnp.float32),
        pltpu.VMEM((7, _LW), jnp.float32),
        pltpu.VMEM((4, _LW), jnp.float32),
        pltpu.VMEM((_LW,), jnp.float32),
        pltpu.VMEM((2, _LW), jnp.float32),
        pltpu.VMEM((7, _LW), jnp.float32),
        pltpu.VMEM((4, _LW), jnp.float32),
        pltpu.VMEM((7, _LW), jnp.float32),
        pltpu.VMEM((4, _LW), jnp.float32),
        pltpu.VMEM((_LW,), jnp.float32),
        pltpu.VMEM((2, _LW), jnp.float32),
        pltpu.VMEM((7, _LW), jnp.float32),
        pltpu.VMEM((4, _LW), jnp.float32),
        pltpu.SemaphoreType.DMA,
        pltpu.SemaphoreType.DMA,
        pltpu.SemaphoreType.DMA,
    ],
)
def _sc_fwd(pb1, ps1, po1, g1, pb2, ps2, po2, g2, orients, dims,
            ob1, os1, ob2, os2,
            c_vm, d_vm,
            pb_v1, ps_v1, po_v1, g_v1, tb_v1, so_v1,
            pb_v2, ps_v2, po_v2, g_v2, tb_v2, so_v2,
            sem1, sem2, semo):
    wid = lax.axis_index("s") * _NC + lax.axis_index("c")
    pltpu.sync_copy(orients, c_vm.at[pl.ds(8, 2)])
    pltpu.sync_copy(dims, d_vm.at[pl.ds(8, 3)])
    d0 = _broadcast_lane(d_vm, 8)
    d1 = _broadcast_lane(d_vm, 9)
    d2 = _broadcast_lane(d_vm, 10)
    orients_v = (_broadcast_lane(c_vm, 8), _broadcast_lane(c_vm, 9))

    heads = (
        (pb1, ps1, po1, g1, ob1, os1, pb_v1, ps_v1, po_v1, g_v1, tb_v1, so_v1, sem1),
        (pb2, ps2, po2, g2, ob2, os2, pb_v2, ps_v2, po_v2, g_v2, tb_v2, so_v2, sem2),
    )

    def run(l0, nl, ng):
        in_cps = []
        for (pbh, psh, poh, gh, _, _, pb_v, ps_v, po_v, g_v, _, _, sem) in heads:
            in_cps.append([
                pltpu.async_copy(pbh.at[:, pl.ds(l0, nl)], pb_v.at[:, pl.ds(0, nl)], sem),
                pltpu.async_copy(psh.at[:, pl.ds(l0, nl)], ps_v.at[:, pl.ds(0, nl)], sem),
                pltpu.async_copy(poh.at[pl.ds(l0, nl)], po_v.at[pl.ds(0, nl)], sem),
                pltpu.async_copy(gh.at[:, pl.ds(l0, nl)], g_v.at[:, pl.ds(0, nl)], sem),
            ])
        out_cps = []
        for h, (_, _, _, _, obh, osh, pb_v, ps_v, po_v, g_v, tb_v, so_v, _) in enumerate(heads):
            for cp in in_cps[h]:
                cp.wait()

            @plsc.parallel_loop(0, ng * 16, step=16, unroll=4)
            def body(s):
                _do_group(s, pb_v, ps_v, po_v, g_v, tb_v, so_v,
                          orients_v[h], d0, d1, d2)
            out_cps.append(
                pltpu.async_copy(tb_v.at[:, pl.ds(0, nl)], obh.at[:, pl.ds(l0, nl)], semo))
            out_cps.append(
                pltpu.async_copy(so_v.at[:, pl.ds(0, nl)], osh.at[:, pl.ds(l0, nl)], semo))
        for cp in out_cps:
            cp.wait()

    run(128 * jnp.minimum(5 * wid, 152), _LW, _LW // 16)


def kernel(pred_bboxes1, pred_class_scores1, pred_objectness1, pred_bboxes_grid1,
           pred_bboxes2, pred_class_scores2, pred_objectness2, pred_bboxes_grid2,
           anchor_orients, anchor_dims):
    ob1, os1, ob2, os2 = _sc_fwd(
        pred_bboxes1.T, pred_class_scores1.T, pred_objectness1, pred_bboxes_grid1.T,
        pred_bboxes2.T, pred_class_scores2.T, pred_objectness2, pred_bboxes_grid2.T,
        anchor_orients, anchor_dims,
    )
    return (ob1.T, os1.T, ob2.T, os2.T)


# speedup vs baseline: 8.8343x; 1.0420x over previous
import functools

import jax
import jax.numpy as jnp
import numpy as np
from jax import lax
from jax.experimental import pallas as pl
from jax.experimental.pallas import tpu as pltpu
from jax.experimental.pallas import tpu_sc as plsc

_N = 20000
_NC, _NS = 2, 16
_NW = _NC * _NS
_LW = 640

_HALF_PI = np.float32(np.pi / 2.0)


def _broadcast_lane(buf, i):
    return plsc.load_gather(buf, [jnp.full((16,), i, dtype=jnp.int32)])


_mesh = plsc.VectorSubcoreMesh(core_axis_name="c", subcore_axis_name="s")


@functools.partial(
    pl.kernel,
    out_type=[
        jax.ShapeDtypeStruct((7, _N), jnp.float32),
        jax.ShapeDtypeStruct((4, _N), jnp.float32),
        jax.ShapeDtypeStruct((7, _N), jnp.float32),
        jax.ShapeDtypeStruct((4, _N), jnp.float32),
    ],
    mesh=_mesh,
    compiler_params=pltpu.CompilerParams(needs_layout_passes=False),
    scratch_types=[
        pltpu.VMEM((16,), jnp.float32),
        pltpu.VMEM((16,), jnp.float32),
        pltpu.VMEM((7, 2 * _LW), jnp.float32),
        pltpu.VMEM((4, 2 * _LW), jnp.float32),
        pltpu.VMEM((2 * _LW,), jnp.float32),
        pltpu.VMEM((2, 2 * _LW), jnp.float32),
        pltpu.VMEM((7, 2 * _LW), jnp.float32),
        pltpu.VMEM((4, 2 * _LW), jnp.float32),
        pltpu.SemaphoreType.DMA,
        pltpu.SemaphoreType.DMA,
    ],
)
def _sc_fwd(pb1, ps1, po1, g1, pb2, ps2, po2, g2, orients, dims,
            ob1, os1, ob2, os2,
            c_vm, d_vm, pb_v, ps_v, po_v, g_v, tb_v, so_v,
            semi, semo):
    wid = lax.axis_index("s") * _NC + lax.axis_index("c")
    l0 = 128 * jnp.minimum(5 * wid, 152)

    heads = ((pb1, ps1, po1, g1, ob1, os1), (pb2, ps2, po2, g2, ob2, os2))
    in_cps = []
    for h, (pbh, psh, poh, gh, _, _) in enumerate(heads):
        o = h * _LW
        in_cps += [
            pltpu.async_copy(pbh.at[:, pl.ds(l0, _LW)], pb_v.at[:, pl.ds(o, _LW)], semi),
            pltpu.async_copy(psh.at[:, pl.ds(l0, _LW)], ps_v.at[:, pl.ds(o, _LW)], semi),
            pltpu.async_copy(poh.at[pl.ds(l0, _LW)], po_v.at[pl.ds(o, _LW)], semi),
            pltpu.async_copy(gh.at[:, pl.ds(l0, _LW)], g_v.at[:, pl.ds(o, _LW)], semi),
        ]

    pltpu.sync_copy(orients, c_vm.at[pl.ds(8, 2)])
    pltpu.sync_copy(dims, d_vm.at[pl.ds(8, 3)])
    d0 = _broadcast_lane(d_vm, 8)
    d1 = _broadcast_lane(d_vm, 9)
    d2 = _broadcast_lane(d_vm, 10)
    o0 = _broadcast_lane(c_vm, 8)
    o1 = _broadcast_lane(c_vm, 9)

    for cp in in_cps:
        cp.wait()

    @plsc.parallel_loop(0, 2 * _LW, step=16, unroll=4)
    def body(s):
        orient_v = jnp.where(s < _LW, o0, o1)
        po_l = po_v[pl.ds(s, 16)]
        mk = jnp.where(po_l >= 0.9, 1.0, 0.0).astype(jnp.float32)
        x0 = pb_v[0, pl.ds(s, 16)]
        s0 = 1.0 / (1.0 + jnp.exp(-x0))
        tb_v[0, pl.ds(s, 16)] = (orient_v + s0 * _HALF_PI) * mk
        for c in (1, 2):
            x = pb_v[c, pl.ds(s, 16)]
            gv = g_v[c - 1, pl.ds(s, 16)]
            sg = 1.0 / (1.0 + jnp.exp(-x))
            tb_v[c, pl.ds(s, 16)] = (sg + gv + 0.5) * mk
        x3 = pb_v[3, pl.ds(s, 16)]
        s3 = 1.0 / (1.0 + jnp.exp(-x3))
        tb_v[3, pl.ds(s, 16)] = s3 * mk
        for c, dv in ((4, d0), (5, d1), (6, d2)):
            x = pb_v[c, pl.ds(s, 16)]
            tb_v[c, pl.ds(s, 16)] = dv * jnp.exp(x) * mk
        for c in range(4):
            so_v[c, pl.ds(s, 16)] = ps_v[c, pl.ds(s, 16)] * mk

    out_cps = []
    for h, (_, _, _, _, obh, osh) in enumerate(heads):
        o = h * _LW
        out_cps += [
            pltpu.async_copy(tb_v.at[:, pl.ds(o, _LW)], obh.at[:, pl.ds(l0, _LW)], semo),
            pltpu.async_copy(so_v.at[:, pl.ds(o, _LW)], osh.at[:, pl.ds(l0, _LW)], semo),
        ]
    for cp in out_cps:
        cp.wait()


def kernel(pred_bboxes1, pred_class_scores1, pred_objectness1, pred_bboxes_grid1,
           pred_bboxes2, pred_class_scores2, pred_objectness2, pred_bboxes_grid2,
           anchor_orients, anchor_dims):
    ob1, os1, ob2, os2 = _sc_fwd(
        pred_bboxes1.T, pred_class_scores1.T, pred_objectness1, pred_bboxes_grid1.T,
        pred_bboxes2.T, pred_class_scores2.T, pred_objectness2, pred_bboxes_grid2.T,
        anchor_orients, anchor_dims,
    )
    return (ob1.T, os1.T, ob2.T, os2.T)


# speedup vs baseline: 8.8536x; 1.0022x over previous
import functools

import jax
import jax.numpy as jnp
import numpy as np
from jax import lax
from jax.experimental import pallas as pl
from jax.experimental.pallas import tpu as pltpu
from jax.experimental.pallas import tpu_sc as plsc

_N = 20000
_NC, _NS = 2, 16
_NW = _NC * _NS
_LW = 640

_HALF_PI = np.float32(np.pi / 2.0)


def _broadcast_lane(buf, i):
    return plsc.load_gather(buf, [jnp.full((16,), i, dtype=jnp.int32)])


_mesh = plsc.VectorSubcoreMesh(core_axis_name="c", subcore_axis_name="s")


@functools.partial(
    pl.kernel,
    out_type=[
        jax.ShapeDtypeStruct((7, _N), jnp.float32),
        jax.ShapeDtypeStruct((4, _N), jnp.float32),
        jax.ShapeDtypeStruct((7, _N), jnp.float32),
        jax.ShapeDtypeStruct((4, _N), jnp.float32),
    ],
    mesh=_mesh,
    compiler_params=pltpu.CompilerParams(
        needs_layout_passes=False,
        disable_bounds_checks=True,
        disable_semaphore_checks=True,
    ),
    scratch_types=[
        pltpu.VMEM((16,), jnp.float32),
        pltpu.VMEM((16,), jnp.float32),
        pltpu.VMEM((7, 2 * _LW), jnp.float32),
        pltpu.VMEM((4, 2 * _LW), jnp.float32),
        pltpu.VMEM((2 * _LW,), jnp.float32),
        pltpu.VMEM((2, 2 * _LW), jnp.float32),
        pltpu.VMEM((7, 2 * _LW), jnp.float32),
        pltpu.VMEM((4, 2 * _LW), jnp.float32),
        pltpu.SemaphoreType.DMA,
        pltpu.SemaphoreType.DMA,
    ],
)
def _sc_fwd(pb1, ps1, po1, g1, pb2, ps2, po2, g2, orients, dims,
            ob1, os1, ob2, os2,
            c_vm, d_vm, pb_v, ps_v, po_v, g_v, tb_v, so_v,
            semi, semo):
    wid = lax.axis_index("s") * _NC + lax.axis_index("c")
    l0 = 128 * jnp.minimum(5 * wid, 152)

    heads = ((pb1, ps1, po1, g1, ob1, os1), (pb2, ps2, po2, g2, ob2, os2))
    in_cps = []
    for h, (pbh, psh, poh, gh, _, _) in enumerate(heads):
        o = h * _LW
        in_cps += [
            pltpu.async_copy(pbh.at[:, pl.ds(l0, _LW)], pb_v.at[:, pl.ds(o, _LW)], semi),
            pltpu.async_copy(psh.at[:, pl.ds(l0, _LW)], ps_v.at[:, pl.ds(o, _LW)], semi),
            pltpu.async_copy(poh.at[pl.ds(l0, _LW)], po_v.at[pl.ds(o, _LW)], semi),
            pltpu.async_copy(gh.at[:, pl.ds(l0, _LW)], g_v.at[:, pl.ds(o, _LW)], semi),
        ]

    cc = pltpu.async_copy(orients, c_vm.at[pl.ds(8, 2)], semi)
    dc = pltpu.async_copy(dims, d_vm.at[pl.ds(8, 3)], semi)
    cc.wait()
    dc.wait()
    d0 = _broadcast_lane(d_vm, 8)
    d1 = _broadcast_lane(d_vm, 9)
    d2 = _broadcast_lane(d_vm, 10)
    o0 = _broadcast_lane(c_vm, 8)
    o1 = _broadcast_lane(c_vm, 9)

    for cp in in_cps:
        cp.wait()

    @plsc.parallel_loop(0, 2 * _LW, step=16, unroll=4)
    def body(s):
        orient_v = jnp.where(s < _LW, o0, o1)
        po_l = po_v[pl.ds(s, 16)]
        mk = jnp.where(po_l >= 0.9, 1.0, 0.0).astype(jnp.float32)
        x0 = pb_v[0, pl.ds(s, 16)]
        s0 = 1.0 / (1.0 + jnp.exp(-x0))
        tb_v[0, pl.ds(s, 16)] = (orient_v + s0 * _HALF_PI) * mk
        for c in (1, 2):
            x = pb_v[c, pl.ds(s, 16)]
            gv = g_v[c - 1, pl.ds(s, 16)]
            sg = 1.0 / (1.0 + jnp.exp(-x))
            tb_v[c, pl.ds(s, 16)] = (sg + gv + 0.5) * mk
        x3 = pb_v[3, pl.ds(s, 16)]
        s3 = 1.0 / (1.0 + jnp.exp(-x3))
        tb_v[3, pl.ds(s, 16)] = s3 * mk
        for c, dv in ((4, d0), (5, d1), (6, d2)):
            x = pb_v[c, pl.ds(s, 16)]
            tb_v[c, pl.ds(s, 16)] = dv * jnp.exp(x) * mk
        for c in range(4):
            so_v[c, pl.ds(s, 16)] = ps_v[c, pl.ds(s, 16)] * mk

    out_cps = []
    for h, (_, _, _, _, obh, osh) in enumerate(heads):
        o = h * _LW
        out_cps += [
            pltpu.async_copy(tb_v.at[:, pl.ds(o, _LW)], obh.at[:, pl.ds(l0, _LW)], semo),
            pltpu.async_copy(so_v.at[:, pl.ds(o, _LW)], osh.at[:, pl.ds(l0, _LW)], semo),
        ]
    for cp in out_cps:
        cp.wait()


def kernel(pred_bboxes1, pred_class_scores1, pred_objectness1, pred_bboxes_grid1,
           pred_bboxes2, pred_class_scores2, pred_objectness2, pred_bboxes_grid2,
           anchor_orients, anchor_dims):
    ob1, os1, ob2, os2 = _sc_fwd(
        pred_bboxes1.T, pred_class_scores1.T, pred_objectness1, pred_bboxes_grid1.T,
        pred_bboxes2.T, pred_class_scores2.T, pred_objectness2, pred_bboxes_grid2.T,
        anchor_orients, anchor_dims,
    )
    return (ob1.T, os1.T, ob2.T, os2.T)


# speedup vs baseline: 8.8955x; 1.0047x over previous
import functools

import jax
import jax.numpy as jnp
import numpy as np
from jax import lax
from jax.experimental import pallas as pl
from jax.experimental.pallas import tpu as pltpu
from jax.experimental.pallas import tpu_sc as plsc

_N = 20000
_NC, _NS = 2, 16
_NW = _NC * _NS
_LW = 640

_HALF_PI = np.float32(np.pi / 2.0)


def _broadcast_lane(buf, i):
    return plsc.load_gather(buf, [jnp.full((16,), i, dtype=jnp.int32)])


_mesh = plsc.VectorSubcoreMesh(core_axis_name="c", subcore_axis_name="s")


@functools.partial(
    pl.kernel,
    out_type=[
        jax.ShapeDtypeStruct((7, _N), jnp.float32),
        jax.ShapeDtypeStruct((4, _N), jnp.float32),
        jax.ShapeDtypeStruct((7, _N), jnp.float32),
        jax.ShapeDtypeStruct((4, _N), jnp.float32),
    ],
    mesh=_mesh,
    compiler_params=pltpu.CompilerParams(needs_layout_passes=False),
    scratch_types=[
        pltpu.VMEM((16,), jnp.float32),
        pltpu.VMEM((16,), jnp.float32),
        pltpu.VMEM((7, 2 * _LW), jnp.float32),
        pltpu.VMEM((4, 2 * _LW), jnp.float32),
        pltpu.VMEM((2 * _LW,), jnp.float32),
        pltpu.VMEM((2, 2 * _LW), jnp.float32),
        pltpu.VMEM((7, 2 * _LW), jnp.float32),
        pltpu.VMEM((4, 2 * _LW), jnp.float32),
        pltpu.SemaphoreType.DMA,
        pltpu.SemaphoreType.DMA,
    ],
)
def _sc_fwd(pb1, ps1, po1, g1, pb2, ps2, po2, g2, orients, dims,
            ob1, os1, ob2, os2,
            c_vm, d_vm, pb_v, ps_v, po_v, g_v, tb_v, so_v,
            semi, semo):
    wid = lax.axis_index("s") * _NC + lax.axis_index("c")
    l0 = 128 * jnp.minimum(5 * wid, 152)

    heads = ((pb1, ps1, po1, g1, ob1, os1), (pb2, ps2, po2, g2, ob2, os2))
    in_cps = []
    for h, (pbh, psh, poh, gh, _, _) in enumerate(heads):
        o = h * _LW
        in_cps += [
            pltpu.async_copy(pbh.at[:, pl.ds(l0, _LW)], pb_v.at[:, pl.ds(o, _LW)], semi),
            pltpu.async_copy(psh.at[:, pl.ds(l0, _LW)], ps_v.at[:, pl.ds(o, _LW)], semi),
            pltpu.async_copy(poh.at[pl.ds(l0, _LW)], po_v.at[pl.ds(o, _LW)], semi),
            pltpu.async_copy(gh.at[:, pl.ds(l0, _LW)], g_v.at[:, pl.ds(o, _LW)], semi),
        ]

    pltpu.sync_copy(orients, c_vm.at[pl.ds(8, 2)])
    pltpu.sync_copy(dims, d_vm.at[pl.ds(8, 3)])
    d0 = _broadcast_lane(d_vm, 8)
    d1 = _broadcast_lane(d_vm, 9)
    d2 = _broadcast_lane(d_vm, 10)
    o0 = _broadcast_lane(c_vm, 8)
    o1 = _broadcast_lane(c_vm, 9)

    for cp in in_cps:
        cp.wait()

    @plsc.parallel_loop(0, 2 * _LW, step=16, unroll=2)
    def body(s):
        orient_v = jnp.where(s < _LW, o0, o1)
        po_l = po_v[pl.ds(s, 16)]
        mk = jnp.where(po_l >= 0.9, 1.0, 0.0).astype(jnp.float32)
        x0 = pb_v[0, pl.ds(s, 16)]
        s0 = 1.0 / (1.0 + jnp.exp(-x0))
        tb_v[0, pl.ds(s, 16)] = (orient_v + s0 * _HALF_PI) * mk
        for c in (1, 2):
            x = pb_v[c, pl.ds(s, 16)]
            gv = g_v[c - 1, pl.ds(s, 16)]
            sg = 1.0 / (1.0 + jnp.exp(-x))
            tb_v[c, pl.ds(s, 16)] = (sg + gv + 0.5) * mk
        x3 = pb_v[3, pl.ds(s, 16)]
        s3 = 1.0 / (1.0 + jnp.exp(-x3))
        tb_v[3, pl.ds(s, 16)] = s3 * mk
        for c, dv in ((4, d0), (5, d1), (6, d2)):
            x = pb_v[c, pl.ds(s, 16)]
            tb_v[c, pl.ds(s, 16)] = dv * jnp.exp(x) * mk
        for c in range(4):
            so_v[c, pl.ds(s, 16)] = ps_v[c, pl.ds(s, 16)] * mk

    out_cps = []
    for h, (_, _, _, _, obh, osh) in enumerate(heads):
        o = h * _LW
        out_cps += [
            pltpu.async_copy(tb_v.at[:, pl.ds(o, _LW)], obh.at[:, pl.ds(l0, _LW)], semo),
            pltpu.async_copy(so_v.at[:, pl.ds(o, _LW)], osh.at[:, pl.ds(l0, _LW)], semo),
        ]
    for cp in out_cps:
        cp.wait()


def kernel(pred_bboxes1, pred_class_scores1, pred_objectness1, pred_bboxes_grid1,
           pred_bboxes2, pred_class_scores2, pred_objectness2, pred_bboxes_grid2,
           anchor_orients, anchor_dims):
    ob1, os1, ob2, os2 = _sc_fwd(
        pred_bboxes1.T, pred_class_scores1.T, pred_objectness1, pred_bboxes_grid1.T,
        pred_bboxes2.T, pred_class_scores2.T, pred_objectness2, pred_bboxes_grid2.T,
        anchor_orients, anchor_dims,
    )
    return (ob1.T, os1.T, ob2.T, os2.T)
